# Initial kernel scaffold; baseline (speedup 1.0000x reference)
import jax
import jax.numpy as jnp
from jax.experimental import pallas as pl


def kernel(x, edge_index, batch, W1, b1, W2, b2, Wc, bc):
    raise NotImplementedError("write your pallas kernel here")


# speedup vs baseline: 1.9360x; 1.9360x over previous
import jax
import jax.numpy as jnp
from jax.experimental import pallas as pl


def _noop_body(x_ref, o_ref):
    o_ref[...] = x_ref[...]


def kernel(x, edge_index, batch, W1, b1, W2, b2, Wc, bc):
    n = x.shape[0]
    src, dst = edge_index[0], edge_index[1]
    deg = jnp.zeros((n,), jnp.float32).at[dst].add(1.0) + 1.0
    s = jax.lax.rsqrt(deg)

    def layer(h, W, b):
        t = h * s[:, None]
        agg = jnp.zeros_like(t).at[dst].add(t[src]) + t
        p = agg * s[:, None]
        return jax.nn.relu(p @ W + b)

    h1 = layer(x, W1, b1)
    h2 = layer(h1, W2, b2)
    seg = jax.ops.segment_sum(h2, batch, num_segments=64)
    cnt = jax.ops.segment_sum(jnp.ones((n, 1), jnp.float32), batch, num_segments=64)
    pooled = seg / jnp.maximum(cnt, 1.0)
    out = pooled @ Wc + bc
    out = pl.pallas_call(
        _noop_body, out_shape=jax.ShapeDtypeStruct(out.shape, out.dtype)
    )(out)
    return out


# speedup vs baseline: 17.8649x; 9.2275x over previous
import functools

import jax
import jax.numpy as jnp
from jax import lax
from jax.experimental import pallas as pl
from jax.experimental.pallas import tpu as pltpu
from jax.experimental.pallas import tpu_sc as plsc

N = 100000
E = 3200000
G = 64
D_IN = 20
D_P = 32
D_H = 128
D_OUT = 30954
L = 16
NSC = 2
NSUB = 16
EG = E // 128
GE = 8
NGROUPS = EG // GE
RT = N // NSUB


def _sc_mesh():
    return plsc.VectorSubcoreMesh(core_axis_name="c", subcore_axis_name="s")


_SC_PARAMS = pltpu.CompilerParams(use_tc_tiling_on_sc=False)


def _make_prop(nchunks):
    npc = nchunks // NSC

    @functools.partial(
        pl.kernel,
        out_type=jax.ShapeDtypeStruct((N, nchunks * L), jnp.float32),
        mesh=_sc_mesh(),
        scratch_types=[
            pltpu.VMEM_SHARED((N, L), jnp.float32),
            pltpu.VMEM((GE, 128), jnp.int32),
            pltpu.VMEM((GE, 128), jnp.int32),
            pltpu.VMEM((GE, 128), jnp.int32),
            pltpu.VMEM((GE * 128, L), jnp.float32),
            pltpu.SemaphoreType.DMA,
        ],
        compiler_params=_SC_PARAMS,
    )
    def prop(hv, src2d, dst2d, zrow, out, acc, sbuf, dbuf, gbuf, rows, sem):
        c = lax.axis_index("c")
        sid = lax.axis_index("s")
        r0 = sid * RT
        for ci in range(npc):
            chunk = c * npc + ci
            pltpu.sync_copy(zrow, acc.at[pl.ds(r0, RT)])
            plsc.subcore_barrier()

            @pl.loop(sid, NGROUPS, step=NSUB)
            def _(g):
                pltpu.sync_copy(src2d.at[pl.ds(g * GE, GE)], sbuf)
                pltpu.sync_copy(dst2d.at[pl.ds(g * GE, GE)], dbuf)
                for j in range(GE):
                    for v in range(128 // L):
                        sl = pl.ds(v * L, L)
                        gbuf[j, sl] = sbuf[j, sl] * nchunks + chunk
                descs = [
                    pltpu.async_copy(
                        hv.at[gbuf.at[j]], rows.at[pl.ds(j * 128, 128)], sem
                    )
                    for j in range(GE)
                ]
                for dsc in descs:
                    dsc.wait()
                for j in range(GE):
                    pltpu.sync_copy(
                        rows.at[pl.ds(j * 128, 128)], acc.at[dbuf.at[j]], add=True
                    )

            plsc.subcore_barrier()
            pltpu.sync_copy(
                acc.at[pl.ds(r0, RT)],
                out.at[pl.ds(r0, RT), pl.ds(chunk * L, L)],
            )

    return prop


_prop2 = _make_prop(2)
_prop8 = _make_prop(8)


@functools.partial(
    pl.kernel,
    out_type=jax.ShapeDtypeStruct((NSC, N, 1), jnp.float32),
    mesh=_sc_mesh(),
    scratch_types=[
        pltpu.VMEM_SHARED((N, 1), jnp.float32),
        pltpu.VMEM((GE, 128), jnp.int32),
        pltpu.VMEM((128, 1), jnp.float32),
    ],
    compiler_params=_SC_PARAMS,
)
def _hist(dst2d, zcol, onesr, out, acc1, dbuf, onesb):
    c = lax.axis_index("c")
    sid = lax.axis_index("s")
    r0 = sid * RT
    pltpu.sync_copy(zcol, acc1.at[pl.ds(r0, RT)])
    pltpu.sync_copy(onesr, onesb)
    plsc.subcore_barrier()

    @pl.loop(c * NSUB + sid, NGROUPS, step=NSC * NSUB)
    def _(g):
        pltpu.sync_copy(dst2d.at[pl.ds(g * GE, GE)], dbuf)
        for j in range(GE):
            pltpu.sync_copy(onesb, acc1.at[dbuf.at[j]], add=True)

    plsc.subcore_barrier()
    pltpu.sync_copy(acc1.at[pl.ds(r0, RT)], out.at[c, pl.ds(r0, RT)])


_BN1 = 2000


def _k1_body(x32_ref, d0_ref, d1_ref, t1_ref, sv_ref):
    deg = d0_ref[0] + d1_ref[0] + 1.0
    s = lax.rsqrt(deg)
    sv_ref[...] = s
    t1_ref[...] = x32_ref[...] * s


def _k1(x32, degp):
    return pl.pallas_call(
        _k1_body,
        grid=(N // _BN1,),
        in_specs=[
            pl.BlockSpec((_BN1, D_P), lambda i: (i, 0)),
            pl.BlockSpec((1, _BN1, 1), lambda i: (0, i, 0)),
            pl.BlockSpec((1, _BN1, 1), lambda i: (1, i, 0)),
        ],
        out_specs=[
            pl.BlockSpec((_BN1, D_P), lambda i: (i, 0)),
            pl.BlockSpec((_BN1, 1), lambda i: (i, 0)),
        ],
        out_shape=[
            jax.ShapeDtypeStruct((N, D_P), jnp.float32),
            jax.ShapeDtypeStruct((N, 1), jnp.float32),
        ],
    )(x32, degp, degp)


_BN2 = 2000


def _k2_body(agg_ref, t1_ref, sv_ref, w_ref, b_ref, t2_ref):
    s = sv_ref[...]
    p = (agg_ref[...] + t1_ref[...]) * s
    h = jnp.maximum(
        jnp.dot(p, w_ref[...], preferred_element_type=jnp.float32) + b_ref[...],
        0.0,
    )
    t2_ref[...] = h * s


def _k2(agg1, t1, sv, W1p, b1r):
    return pl.pallas_call(
        _k2_body,
        grid=(N // _BN2,),
        in_specs=[
            pl.BlockSpec((_BN2, D_P), lambda i: (i, 0)),
            pl.BlockSpec((_BN2, D_P), lambda i: (i, 0)),
            pl.BlockSpec((_BN2, 1), lambda i: (i, 0)),
            pl.BlockSpec((D_P, D_H), lambda i: (0, 0)),
            pl.BlockSpec((1, D_H), lambda i: (0, 0)),
        ],
        out_specs=pl.BlockSpec((_BN2, D_H), lambda i: (i, 0)),
        out_shape=jax.ShapeDtypeStruct((N, D_H), jnp.float32),
    )(agg1, t1, sv, W1p, b1r)


def _k3_body(agg_ref, t2_ref, sv_ref, bat_ref, w_ref, b_ref, out_ref, pacc, cacc):
    i = pl.program_id(0)

    @pl.when(i == 0)
    def _():
        pacc[...] = jnp.zeros_like(pacc)
        cacc[...] = jnp.zeros_like(cacc)

    s = sv_ref[...]
    p = (agg_ref[...] + t2_ref[...]) * s
    h = jnp.maximum(
        jnp.dot(p, w_ref[...], preferred_element_type=jnp.float32) + b_ref[...],
        0.0,
    )
    gids = lax.broadcasted_iota(jnp.int32, (G, _BN2), 0)
    onehot = jnp.where(gids == bat_ref[0], 1.0, 0.0)
    pacc[...] += jnp.dot(onehot, h, preferred_element_type=jnp.float32)
    cacc[...] += jnp.sum(onehot, axis=1, keepdims=True)

    @pl.when(i == pl.num_programs(0) - 1)
    def _():
        out_ref[...] = pacc[...] / jnp.maximum(cacc[...], 1.0)


def _k3(agg2, t2, sv, batchr, W2, b2r):
    return pl.pallas_call(
        _k3_body,
        grid=(N // _BN2,),
        in_specs=[
            pl.BlockSpec((_BN2, D_H), lambda i: (i, 0)),
            pl.BlockSpec((_BN2, D_H), lambda i: (i, 0)),
            pl.BlockSpec((_BN2, 1), lambda i: (i, 0)),
            pl.BlockSpec((1, 1, _BN2), lambda i: (i, 0, 0)),
            pl.BlockSpec((D_H, D_H), lambda i: (0, 0)),
            pl.BlockSpec((1, D_H), lambda i: (0, 0)),
        ],
        out_specs=pl.BlockSpec((G, D_H), lambda i: (0, 0)),
        out_shape=jax.ShapeDtypeStruct((G, D_H), jnp.float32),
        scratch_shapes=[
            pltpu.VMEM((G, D_H), jnp.float32),
            pltpu.VMEM((G, 1), jnp.float32),
        ],
    )(agg2, t2, sv, batchr, W2, b2r)


def _k4_body(p_ref, wc_ref, bc_ref, out_ref):
    out_ref[...] = (
        jnp.dot(p_ref[...], wc_ref[...], preferred_element_type=jnp.float32)
        + bc_ref[...]
    )


def _k4(pooled, Wc, bcr):
    return pl.pallas_call(
        _k4_body,
        out_shape=jax.ShapeDtypeStruct((G, D_OUT), jnp.float32),
    )(pooled, Wc, bcr)


def kernel(x, edge_index, batch, W1, b1, W2, b2, Wc, bc):
    src2d = edge_index[0].reshape(EG, 128)
    dst2d = edge_index[1].reshape(EG, 128)
    x32 = jnp.pad(x, ((0, 0), (0, D_P - D_IN)))
    W1p = jnp.pad(W1, ((0, D_P - D_IN), (0, 0)))
    b1r = b1.reshape(1, D_H)
    b2r = b2.reshape(1, D_H)
    bcr = bc.reshape(1, D_OUT)
    batchr = batch.reshape(N // _BN2, 1, _BN2)
    zrow = jnp.zeros((RT, L), jnp.float32)
    zcol = jnp.zeros((RT, 1), jnp.float32)
    onesr = jnp.ones((128, 1), jnp.float32)

    degp = _hist(dst2d, zcol, onesr)
    t1, sv = _k1(x32, degp)
    agg1 = _prop2(t1.reshape(N * 2, L), src2d, dst2d, zrow)
    t2 = _k2(agg1, t1, sv, W1p, b1r)
    agg2 = _prop8(t2.reshape(N * 8, L), src2d, dst2d, zrow)
    pooled = _k3(agg2, t2, sv, batchr, W2, b2r)
    return _k4(pooled, Wc, bcr)


# speedup vs baseline: 27.8418x; 1.5585x over previous
import functools

import jax
import jax.numpy as jnp
from jax import lax
from jax.experimental import pallas as pl
from jax.experimental.pallas import tpu as pltpu
from jax.experimental.pallas import tpu_sc as plsc

N = 100000
E = 3200000
G = 64
D_IN = 20
D_P = 32
D_H = 128
D_OUT = 30954
L = 16
NSC = 2
NSUB = 16
EG = E // 128
GE = 5
NGROUPS = EG // GE
GEH = 8
RT = N // NSUB


def _sc_mesh():
    return plsc.VectorSubcoreMesh(core_axis_name="c", subcore_axis_name="s")


_SC_PARAMS = pltpu.CompilerParams(use_tc_tiling_on_sc=False)


def _make_prop(nchunks):
    npc = nchunks // NSC

    @functools.partial(
        pl.kernel,
        out_type=jax.ShapeDtypeStruct((N, nchunks * L), jnp.float32),
        mesh=_sc_mesh(),
        scratch_types=[
            pltpu.VMEM_SHARED((N, L), jnp.float32),
            pltpu.VMEM((GE, 128), jnp.int32),
            pltpu.VMEM((GE, 128), jnp.int32),
            pltpu.VMEM((GE, 128), jnp.int32),
            pltpu.VMEM((GE, 128), jnp.int32),
            pltpu.VMEM((GE * 128, L), jnp.float32),
            pltpu.VMEM((GE * 128, L), jnp.float32),
            pltpu.SemaphoreType.DMA,
            pltpu.SemaphoreType.DMA,
            pltpu.SemaphoreType.DMA,
            pltpu.SemaphoreType.DMA,
        ],
        compiler_params=_SC_PARAMS,
    )
    def prop(hv, src2d, dst2d, zrow, out, acc,
             sbufA, dbufA, sbufB, dbufB, rowsA, rowsB,
             semE, semGA, semGB, semS):
        c = lax.axis_index("c")
        sid = lax.axis_index("s")
        r0 = sid * RT
        a = sid * NGROUPS // NSUB
        b = (sid + 1) * NGROUPS // NSUB
        npairs = (b - a) // 2
        odd = (b - a) - 2 * npairs

        def compute_gidx(sbuf, chunk):
            for j in range(GE):
                for v in range(128 // L):
                    sl = pl.ds(v * L, L)
                    sbuf[j, sl] = sbuf[j, sl] * nchunks + chunk

        def fire_gathers(gbuf, rows, sem):
            return [
                pltpu.async_copy(
                    hv.at[gbuf.at[j]], rows.at[pl.ds(j * 128, 128)], sem
                )
                for j in range(GE)
            ]

        def fire_scatters(rows, dbuf):
            return [
                pltpu.async_copy(
                    rows.at[pl.ds(j * 128, 128)], acc.at[dbuf.at[j]], semS,
                    add=True,
                )
                for j in range(GE)
            ]

        for ci in range(npc):
            chunk = c * npc + ci
            pltpu.sync_copy(zrow, acc.at[pl.ds(r0, RT)])
            plsc.subcore_barrier()

            @pl.loop(0, npairs)
            def _(k):
                gA = a + 2 * k
                gB = gA + 1
                eA = [
                    pltpu.async_copy(src2d.at[pl.ds(gA * GE, GE)], sbufA, semE),
                    pltpu.async_copy(dst2d.at[pl.ds(gA * GE, GE)], dbufA, semE),
                ]
                eB = [
                    pltpu.async_copy(src2d.at[pl.ds(gB * GE, GE)], sbufB, semE),
                    pltpu.async_copy(dst2d.at[pl.ds(gB * GE, GE)], dbufB, semE),
                ]
                for d_ in eA:
                    d_.wait()
                compute_gidx(sbufA, chunk)
                descA = fire_gathers(sbufA, rowsA, semGA)
                for d_ in eB:
                    d_.wait()
                compute_gidx(sbufB, chunk)
                descB = fire_gathers(sbufB, rowsB, semGB)
                for d_ in descA:
                    d_.wait()
                scA = fire_scatters(rowsA, dbufA)
                for d_ in descB:
                    d_.wait()
                scB = fire_scatters(rowsB, dbufB)
                for d_ in scA + scB:
                    d_.wait()

            @pl.when(odd > 0)
            def _():
                g = b - 1
                pltpu.sync_copy(src2d.at[pl.ds(g * GE, GE)], sbufA)
                pltpu.sync_copy(dst2d.at[pl.ds(g * GE, GE)], dbufA)
                compute_gidx(sbufA, chunk)
                for d_ in fire_gathers(sbufA, rowsA, semGA):
                    d_.wait()
                for d_ in fire_scatters(rowsA, dbufA):
                    d_.wait()

            plsc.subcore_barrier()
            pltpu.sync_copy(
                acc.at[pl.ds(r0, RT)],
                out.at[pl.ds(r0, RT), pl.ds(chunk * L, L)],
            )

    return prop


_prop2 = _make_prop(2)
_prop8 = _make_prop(8)


@functools.partial(
    pl.kernel,
    out_type=jax.ShapeDtypeStruct((NSC, N, 1), jnp.float32),
    mesh=_sc_mesh(),
    scratch_types=[
        pltpu.VMEM_SHARED((N, 1), jnp.float32),
        pltpu.VMEM((GEH, 128), jnp.int32),
        pltpu.VMEM((128, 1), jnp.float32),
    ],
    compiler_params=_SC_PARAMS,
)
def _hist(dst2d, zcol, onesr, out, acc1, dbuf, onesb):
    c = lax.axis_index("c")
    sid = lax.axis_index("s")
    r0 = sid * RT
    pltpu.sync_copy(zcol, acc1.at[pl.ds(r0, RT)])
    pltpu.sync_copy(onesr, onesb)
    plsc.subcore_barrier()

    @pl.loop(c * NSUB + sid, EG // GEH, step=NSC * NSUB)
    def _(g):
        pltpu.sync_copy(dst2d.at[pl.ds(g * GEH, GEH)], dbuf)
        for j in range(GEH):
            pltpu.sync_copy(onesb, acc1.at[dbuf.at[j]], add=True)

    plsc.subcore_barrier()
    pltpu.sync_copy(acc1.at[pl.ds(r0, RT)], out.at[c, pl.ds(r0, RT)])


_BN1 = 2000


def _k1_body(x32_ref, d0_ref, d1_ref, t1_ref, sv_ref):
    deg = d0_ref[0] + d1_ref[0] + 1.0
    s = lax.rsqrt(deg)
    sv_ref[...] = s
    t1_ref[...] = x32_ref[...] * s


def _k1(x32, degp):
    return pl.pallas_call(
        _k1_body,
        grid=(N // _BN1,),
        in_specs=[
            pl.BlockSpec((_BN1, D_P), lambda i: (i, 0)),
            pl.BlockSpec((1, _BN1, 1), lambda i: (0, i, 0)),
            pl.BlockSpec((1, _BN1, 1), lambda i: (1, i, 0)),
        ],
        out_specs=[
            pl.BlockSpec((_BN1, D_P), lambda i: (i, 0)),
            pl.BlockSpec((_BN1, 1), lambda i: (i, 0)),
        ],
        out_shape=[
            jax.ShapeDtypeStruct((N, D_P), jnp.float32),
            jax.ShapeDtypeStruct((N, 1), jnp.float32),
        ],
    )(x32, degp, degp)


_BN2 = 2000


def _k2_body(agg_ref, t1_ref, sv_ref, w_ref, b_ref, t2_ref):
    s = sv_ref[...]
    p = (agg_ref[...] + t1_ref[...]) * s
    h = jnp.maximum(
        jnp.dot(p, w_ref[...], preferred_element_type=jnp.float32) + b_ref[...],
        0.0,
    )
    t2_ref[...] = h * s


def _k2(agg1, t1, sv, W1p, b1r):
    return pl.pallas_call(
        _k2_body,
        grid=(N // _BN2,),
        in_specs=[
            pl.BlockSpec((_BN2, D_P), lambda i: (i, 0)),
            pl.BlockSpec((_BN2, D_P), lambda i: (i, 0)),
            pl.BlockSpec((_BN2, 1), lambda i: (i, 0)),
            pl.BlockSpec((D_P, D_H), lambda i: (0, 0)),
            pl.BlockSpec((1, D_H), lambda i: (0, 0)),
        ],
        out_specs=pl.BlockSpec((_BN2, D_H), lambda i: (i, 0)),
        out_shape=jax.ShapeDtypeStruct((N, D_H), jnp.float32),
    )(agg1, t1, sv, W1p, b1r)


def _k3_body(agg_ref, t2_ref, sv_ref, bat_ref, w_ref, b_ref, out_ref, pacc, cacc):
    i = pl.program_id(0)

    @pl.when(i == 0)
    def _():
        pacc[...] = jnp.zeros_like(pacc)
        cacc[...] = jnp.zeros_like(cacc)

    s = sv_ref[...]
    p = (agg_ref[...] + t2_ref[...]) * s
    h = jnp.maximum(
        jnp.dot(p, w_ref[...], preferred_element_type=jnp.float32) + b_ref[...],
        0.0,
    )
    gids = lax.broadcasted_iota(jnp.int32, (G, _BN2), 0)
    onehot = jnp.where(gids == bat_ref[0], 1.0, 0.0)
    pacc[...] += jnp.dot(onehot, h, preferred_element_type=jnp.float32)
    cacc[...] += jnp.sum(onehot, axis=1, keepdims=True)

    @pl.when(i == pl.num_programs(0) - 1)
    def _():
        out_ref[...] = pacc[...] / jnp.maximum(cacc[...], 1.0)


def _k3(agg2, t2, sv, batchr, W2, b2r):
    return pl.pallas_call(
        _k3_body,
        grid=(N // _BN2,),
        in_specs=[
            pl.BlockSpec((_BN2, D_H), lambda i: (i, 0)),
            pl.BlockSpec((_BN2, D_H), lambda i: (i, 0)),
            pl.BlockSpec((_BN2, 1), lambda i: (i, 0)),
            pl.BlockSpec((1, 1, _BN2), lambda i: (i, 0, 0)),
            pl.BlockSpec((D_H, D_H), lambda i: (0, 0)),
            pl.BlockSpec((1, D_H), lambda i: (0, 0)),
        ],
        out_specs=pl.BlockSpec((G, D_H), lambda i: (0, 0)),
        out_shape=jax.ShapeDtypeStruct((G, D_H), jnp.float32),
        scratch_shapes=[
            pltpu.VMEM((G, D_H), jnp.float32),
            pltpu.VMEM((G, 1), jnp.float32),
        ],
    )(agg2, t2, sv, batchr, W2, b2r)


def _k4_body(p_ref, wc_ref, bc_ref, out_ref):
    out_ref[...] = (
        jnp.dot(p_ref[...], wc_ref[...], preferred_element_type=jnp.float32)
        + bc_ref[...]
    )


def _k4(pooled, Wc, bcr):
    return pl.pallas_call(
        _k4_body,
        out_shape=jax.ShapeDtypeStruct((G, D_OUT), jnp.float32),
    )(pooled, Wc, bcr)


def kernel(x, edge_index, batch, W1, b1, W2, b2, Wc, bc):
    src2d = edge_index[0].reshape(EG, 128)
    dst2d = edge_index[1].reshape(EG, 128)
    x32 = jnp.pad(x, ((0, 0), (0, D_P - D_IN)))
    W1p = jnp.pad(W1, ((0, D_P - D_IN), (0, 0)))
    b1r = b1.reshape(1, D_H)
    b2r = b2.reshape(1, D_H)
    bcr = bc.reshape(1, D_OUT)
    batchr = batch.reshape(N // _BN2, 1, _BN2)
    zrow = jnp.zeros((RT, L), jnp.float32)
    zcol = jnp.zeros((RT, 1), jnp.float32)
    onesr = jnp.ones((128, 1), jnp.float32)

    degp = _hist(dst2d, zcol, onesr)
    t1, sv = _k1(x32, degp)
    agg1 = _prop2(t1.reshape(N * 2, L), src2d, dst2d, zrow)
    t2 = _k2(agg1, t1, sv, W1p, b1r)
    agg2 = _prop8(t2.reshape(N * 8, L), src2d, dst2d, zrow)
    pooled = _k3(agg2, t2, sv, batchr, W2, b2r)
    return _k4(pooled, Wc, bcr)


# speedup vs baseline: 32.8778x; 1.1809x over previous
import functools

import jax
import jax.numpy as jnp
from jax import lax
from jax.experimental import pallas as pl
from jax.experimental.pallas import tpu as pltpu
from jax.experimental.pallas import tpu_sc as plsc

N = 100000
E = 3200000
G = 64
D_IN = 20
D_P = 32
D_H = 128
D_OUT = 30954
L = 16
NSC = 2
NSUB = 16
EG = E // 128
GE = 5
NGROUPS = EG // GE
GEH = 8
RT = N // NSUB


def _sc_mesh():
    return plsc.VectorSubcoreMesh(core_axis_name="c", subcore_axis_name="s")


_SC_PARAMS = pltpu.CompilerParams(use_tc_tiling_on_sc=False)


def _make_prop(nchunks):
    npc = nchunks // NSC

    @functools.partial(
        pl.kernel,
        out_type=jax.ShapeDtypeStruct((N, nchunks * L), jnp.float32),
        mesh=_sc_mesh(),
        scratch_types=[
            pltpu.VMEM_SHARED((N, L), jnp.float32),
            pltpu.VMEM((GE, 128), jnp.int32),
            pltpu.VMEM((GE, 128), jnp.int32),
            pltpu.VMEM((GE, 128), jnp.int32),
            pltpu.VMEM((GE, 128), jnp.int32),
            pltpu.VMEM((GE * 128, L), jnp.float32),
            pltpu.VMEM((GE * 128, L), jnp.float32),
            pltpu.SemaphoreType.DMA,
            pltpu.SemaphoreType.DMA,
            pltpu.SemaphoreType.DMA,
            pltpu.SemaphoreType.DMA,
        ],
        compiler_params=_SC_PARAMS,
    )
    def prop(hv, src2d, dst2d, zrow, out, acc,
             sbufA, dbufA, sbufB, dbufB, rowsA, rowsB,
             semE, semGA, semGB, semS):
        c = lax.axis_index("c")
        sid = lax.axis_index("s")
        r0 = sid * RT
        a = sid * NGROUPS // NSUB
        b = (sid + 1) * NGROUPS // NSUB
        npairs = (b - a) // 2
        odd = (b - a) - 2 * npairs

        def compute_gidx(sbuf, chunk):
            for j in range(GE):
                for v in range(128 // L):
                    sl = pl.ds(v * L, L)
                    sbuf[j, sl] = sbuf[j, sl] * nchunks + chunk

        def fire_gathers(gbuf, rows, sem):
            return [
                pltpu.async_copy(
                    hv.at[gbuf.at[j]], rows.at[pl.ds(j * 128, 128)], sem
                )
                for j in range(GE)
            ]

        def fire_scatters(rows, dbuf):
            return [
                pltpu.async_copy(
                    rows.at[pl.ds(j * 128, 128)], acc.at[dbuf.at[j]], semS,
                    add=True,
                )
                for j in range(GE)
            ]

        for ci in range(npc):
            chunk = c * npc + ci
            pltpu.sync_copy(zrow, acc.at[pl.ds(r0, RT)])
            plsc.subcore_barrier()

            def drain_scatters():
                for rows, dbuf in ((rowsA, dbufA), (rowsB, dbufB)):
                    for j in range(GE):
                        pltpu.make_async_copy(
                            rows.at[pl.ds(j * 128, 128)], acc.at[dbuf.at[j]],
                            semS,
                        ).wait()

            @pl.loop(0, npairs)
            def _(k):
                gA = a + 2 * k
                gB = gA + 1
                eSA = pltpu.async_copy(src2d.at[pl.ds(gA * GE, GE)], sbufA, semE)
                eSB = pltpu.async_copy(src2d.at[pl.ds(gB * GE, GE)], sbufB, semE)

                @pl.when(k > 0)
                def _():
                    drain_scatters()

                eDA = pltpu.async_copy(dst2d.at[pl.ds(gA * GE, GE)], dbufA, semE)
                eDB = pltpu.async_copy(dst2d.at[pl.ds(gB * GE, GE)], dbufB, semE)
                eSA.wait()
                compute_gidx(sbufA, chunk)
                descA = fire_gathers(sbufA, rowsA, semGA)
                eSB.wait()
                compute_gidx(sbufB, chunk)
                descB = fire_gathers(sbufB, rowsB, semGB)
                eDA.wait()
                eDB.wait()
                for d_ in descA:
                    d_.wait()
                fire_scatters(rowsA, dbufA)
                for d_ in descB:
                    d_.wait()
                fire_scatters(rowsB, dbufB)

            @pl.when(npairs > 0)
            def _():
                drain_scatters()

            @pl.when(odd > 0)
            def _():
                g = b - 1
                pltpu.sync_copy(src2d.at[pl.ds(g * GE, GE)], sbufA)
                pltpu.sync_copy(dst2d.at[pl.ds(g * GE, GE)], dbufA)
                compute_gidx(sbufA, chunk)
                for d_ in fire_gathers(sbufA, rowsA, semGA):
                    d_.wait()
                for d_ in fire_scatters(rowsA, dbufA):
                    d_.wait()

            plsc.subcore_barrier()
            pltpu.sync_copy(
                acc.at[pl.ds(r0, RT)],
                out.at[pl.ds(r0, RT), pl.ds(chunk * L, L)],
            )

    return prop


_prop2 = _make_prop(2)
_prop8 = _make_prop(8)


@functools.partial(
    pl.kernel,
    out_type=jax.ShapeDtypeStruct((NSC, N, 1), jnp.float32),
    mesh=_sc_mesh(),
    scratch_types=[
        pltpu.VMEM_SHARED((N, 1), jnp.float32),
        pltpu.VMEM((GEH, 128), jnp.int32),
        pltpu.VMEM((128, 1), jnp.float32),
        pltpu.SemaphoreType.DMA,
    ],
    compiler_params=_SC_PARAMS,
)
def _hist(dst2d, zcol, onesr, out, acc1, dbuf, onesb, semS):
    c = lax.axis_index("c")
    sid = lax.axis_index("s")
    r0 = sid * RT
    pltpu.sync_copy(zcol, acc1.at[pl.ds(r0, RT)])
    pltpu.sync_copy(onesr, onesb)
    plsc.subcore_barrier()

    w = c * NSUB + sid

    @pl.loop(w, EG // GEH, step=NSC * NSUB)
    def _(g):
        @pl.when(g != w)
        def _():
            for j in range(GEH):
                pltpu.make_async_copy(onesb, acc1.at[dbuf.at[j]], semS).wait()

        pltpu.sync_copy(dst2d.at[pl.ds(g * GEH, GEH)], dbuf)
        for j in range(GEH):
            pltpu.async_copy(onesb, acc1.at[dbuf.at[j]], semS, add=True)

    for j in range(GEH):
        pltpu.make_async_copy(onesb, acc1.at[dbuf.at[j]], semS).wait()

    plsc.subcore_barrier()
    pltpu.sync_copy(acc1.at[pl.ds(r0, RT)], out.at[c, pl.ds(r0, RT)])


_BN1 = 2000


def _k1_body(x32_ref, d0_ref, d1_ref, t1_ref, sv_ref):
    deg = d0_ref[0] + d1_ref[0] + 1.0
    s = lax.rsqrt(deg)
    sv_ref[...] = s
    t1_ref[...] = x32_ref[...] * s


def _k1(x32, degp):
    return pl.pallas_call(
        _k1_body,
        grid=(N // _BN1,),
        in_specs=[
            pl.BlockSpec((_BN1, D_P), lambda i: (i, 0)),
            pl.BlockSpec((1, _BN1, 1), lambda i: (0, i, 0)),
            pl.BlockSpec((1, _BN1, 1), lambda i: (1, i, 0)),
        ],
        out_specs=[
            pl.BlockSpec((_BN1, D_P), lambda i: (i, 0)),
            pl.BlockSpec((_BN1, 1), lambda i: (i, 0)),
        ],
        out_shape=[
            jax.ShapeDtypeStruct((N, D_P), jnp.float32),
            jax.ShapeDtypeStruct((N, 1), jnp.float32),
        ],
    )(x32, degp, degp)


_BN2 = 2000


def _k2_body(agg_ref, t1_ref, sv_ref, w_ref, b_ref, t2_ref):
    s = sv_ref[...]
    p = (agg_ref[...] + t1_ref[...]) * s
    h = jnp.maximum(
        jnp.dot(p, w_ref[...], preferred_element_type=jnp.float32) + b_ref[...],
        0.0,
    )
    t2_ref[...] = h * s


def _k2(agg1, t1, sv, W1p, b1r):
    return pl.pallas_call(
        _k2_body,
        grid=(N // _BN2,),
        in_specs=[
            pl.BlockSpec((_BN2, D_P), lambda i: (i, 0)),
            pl.BlockSpec((_BN2, D_P), lambda i: (i, 0)),
            pl.BlockSpec((_BN2, 1), lambda i: (i, 0)),
            pl.BlockSpec((D_P, D_H), lambda i: (0, 0)),
            pl.BlockSpec((1, D_H), lambda i: (0, 0)),
        ],
        out_specs=pl.BlockSpec((_BN2, D_H), lambda i: (i, 0)),
        out_shape=jax.ShapeDtypeStruct((N, D_H), jnp.float32),
    )(agg1, t1, sv, W1p, b1r)


def _k3_body(agg_ref, t2_ref, sv_ref, bat_ref, w_ref, b_ref, out_ref, pacc, cacc):
    i = pl.program_id(0)

    @pl.when(i == 0)
    def _():
        pacc[...] = jnp.zeros_like(pacc)
        cacc[...] = jnp.zeros_like(cacc)

    s = sv_ref[...]
    p = (agg_ref[...] + t2_ref[...]) * s
    h = jnp.maximum(
        jnp.dot(p, w_ref[...], preferred_element_type=jnp.float32) + b_ref[...],
        0.0,
    )
    gids = lax.broadcasted_iota(jnp.int32, (G, _BN2), 0)
    onehot = jnp.where(gids == bat_ref[0], 1.0, 0.0)
    pacc[...] += jnp.dot(onehot, h, preferred_element_type=jnp.float32)
    cacc[...] += jnp.sum(onehot, axis=1, keepdims=True)

    @pl.when(i == pl.num_programs(0) - 1)
    def _():
        out_ref[...] = pacc[...] / jnp.maximum(cacc[...], 1.0)


def _k3(agg2, t2, sv, batchr, W2, b2r):
    return pl.pallas_call(
        _k3_body,
        grid=(N // _BN2,),
        in_specs=[
            pl.BlockSpec((_BN2, D_H), lambda i: (i, 0)),
            pl.BlockSpec((_BN2, D_H), lambda i: (i, 0)),
            pl.BlockSpec((_BN2, 1), lambda i: (i, 0)),
            pl.BlockSpec((1, 1, _BN2), lambda i: (i, 0, 0)),
            pl.BlockSpec((D_H, D_H), lambda i: (0, 0)),
            pl.BlockSpec((1, D_H), lambda i: (0, 0)),
        ],
        out_specs=pl.BlockSpec((G, D_H), lambda i: (0, 0)),
        out_shape=jax.ShapeDtypeStruct((G, D_H), jnp.float32),
        scratch_shapes=[
            pltpu.VMEM((G, D_H), jnp.float32),
            pltpu.VMEM((G, 1), jnp.float32),
        ],
    )(agg2, t2, sv, batchr, W2, b2r)


def _k4_body(p_ref, wc_ref, bc_ref, out_ref):
    out_ref[...] = (
        jnp.dot(p_ref[...], wc_ref[...], preferred_element_type=jnp.float32)
        + bc_ref[...]
    )


def _k4(pooled, Wc, bcr):
    return pl.pallas_call(
        _k4_body,
        out_shape=jax.ShapeDtypeStruct((G, D_OUT), jnp.float32),
    )(pooled, Wc, bcr)


def kernel(x, edge_index, batch, W1, b1, W2, b2, Wc, bc):
    src2d = edge_index[0].reshape(EG, 128)
    dst2d = edge_index[1].reshape(EG, 128)
    x32 = jnp.pad(x, ((0, 0), (0, D_P - D_IN)))
    W1p = jnp.pad(W1, ((0, D_P - D_IN), (0, 0)))
    b1r = b1.reshape(1, D_H)
    b2r = b2.reshape(1, D_H)
    bcr = bc.reshape(1, D_OUT)
    batchr = batch.reshape(N // _BN2, 1, _BN2)
    zrow = jnp.zeros((RT, L), jnp.float32)
    zcol = jnp.zeros((RT, 1), jnp.float32)
    onesr = jnp.ones((128, 1), jnp.float32)

    degp = _hist(dst2d, zcol, onesr)
    t1, sv = _k1(x32, degp)
    agg1 = _prop2(t1.reshape(N * 2, L), src2d, dst2d, zrow)
    t2 = _k2(agg1, t1, sv, W1p, b1r)
    agg2 = _prop8(t2.reshape(N * 8, L), src2d, dst2d, zrow)
    pooled = _k3(agg2, t2, sv, batchr, W2, b2r)
    return _k4(pooled, Wc, bcr)


# speedup vs baseline: 42.2515x; 1.2851x over previous
import functools

import jax
import jax.numpy as jnp
from jax import lax
from jax.experimental import pallas as pl
from jax.experimental.pallas import tpu as pltpu
from jax.experimental.pallas import tpu_sc as plsc

N = 100000
E = 3200000
G = 64
D_IN = 20
D_P = 32
D_H = 128
D_OUT = 30954
L = 16
NSC = 2
NSUB = 16
EG = E // 128
GE = 5
NGROUPS = EG // GE
GEH = 8
RT = N // NSUB


def _sc_mesh():
    return plsc.VectorSubcoreMesh(core_axis_name="c", subcore_axis_name="s")


_SC_PARAMS = pltpu.CompilerParams(use_tc_tiling_on_sc=False)


def _make_prop(nchunks, width, dtype):
    npc = nchunks // NSC

    @functools.partial(
        pl.kernel,
        out_type=jax.ShapeDtypeStruct((N, nchunks * width), dtype),
        mesh=_sc_mesh(),
        scratch_types=[
            pltpu.VMEM_SHARED((N, width), dtype),
            pltpu.VMEM((GE, 128), jnp.int32),
            pltpu.VMEM((GE, 128), jnp.int32),
            pltpu.VMEM((GE, 128), jnp.int32),
            pltpu.VMEM((GE, 128), jnp.int32),
            pltpu.VMEM((GE * 128, width), dtype),
            pltpu.VMEM((GE * 128, width), dtype),
            pltpu.SemaphoreType.DMA,
            pltpu.SemaphoreType.DMA,
            pltpu.SemaphoreType.DMA,
            pltpu.SemaphoreType.DMA,
        ],
        compiler_params=_SC_PARAMS,
    )
    def prop(hv, src2d, dst2d, zrow, out, acc,
             sbufA, dbufA, sbufB, dbufB, rowsA, rowsB,
             semE, semGA, semGB, semS):
        c = lax.axis_index("c")
        sid = lax.axis_index("s")
        r0 = sid * RT
        a = sid * NGROUPS // NSUB
        b = (sid + 1) * NGROUPS // NSUB
        npairs = (b - a) // 2
        odd = (b - a) - 2 * npairs

        def compute_gidx(sbuf, chunk):
            for j in range(GE):
                for v in range(128 // L):
                    sl = pl.ds(v * L, L)
                    sbuf[j, sl] = sbuf[j, sl] * nchunks + chunk

        def fire_gathers(gbuf, rows, sem):
            return [
                pltpu.async_copy(
                    hv.at[gbuf.at[j]], rows.at[pl.ds(j * 128, 128)], sem
                )
                for j in range(GE)
            ]

        def fire_scatters(rows, dbuf):
            return [
                pltpu.async_copy(
                    rows.at[pl.ds(j * 128, 128)], acc.at[dbuf.at[j]], semS,
                    add=True,
                )
                for j in range(GE)
            ]

        for ci in range(npc):
            chunk = c * npc + ci
            pltpu.sync_copy(zrow, acc.at[pl.ds(r0, RT)])
            plsc.subcore_barrier()

            def drain_scatters():
                for rows, dbuf in ((rowsA, dbufA), (rowsB, dbufB)):
                    for j in range(GE):
                        pltpu.make_async_copy(
                            rows.at[pl.ds(j * 128, 128)], acc.at[dbuf.at[j]],
                            semS,
                        ).wait()

            @pl.loop(0, npairs)
            def _(k):
                gA = a + 2 * k
                gB = gA + 1
                eSA = pltpu.async_copy(src2d.at[pl.ds(gA * GE, GE)], sbufA, semE)
                eSB = pltpu.async_copy(src2d.at[pl.ds(gB * GE, GE)], sbufB, semE)

                @pl.when(k > 0)
                def _():
                    drain_scatters()

                eDA = pltpu.async_copy(dst2d.at[pl.ds(gA * GE, GE)], dbufA, semE)
                eDB = pltpu.async_copy(dst2d.at[pl.ds(gB * GE, GE)], dbufB, semE)
                eSA.wait()
                compute_gidx(sbufA, chunk)
                descA = fire_gathers(sbufA, rowsA, semGA)
                eSB.wait()
                compute_gidx(sbufB, chunk)
                descB = fire_gathers(sbufB, rowsB, semGB)
                eDA.wait()
                eDB.wait()
                for d_ in descA:
                    d_.wait()
                fire_scatters(rowsA, dbufA)
                for d_ in descB:
                    d_.wait()
                fire_scatters(rowsB, dbufB)

            @pl.when(npairs > 0)
            def _():
                drain_scatters()

            @pl.when(odd > 0)
            def _():
                g = b - 1
                pltpu.sync_copy(src2d.at[pl.ds(g * GE, GE)], sbufA)
                pltpu.sync_copy(dst2d.at[pl.ds(g * GE, GE)], dbufA)
                compute_gidx(sbufA, chunk)
                for d_ in fire_gathers(sbufA, rowsA, semGA):
                    d_.wait()
                for d_ in fire_scatters(rowsA, dbufA):
                    d_.wait()

            plsc.subcore_barrier()
            pltpu.sync_copy(
                acc.at[pl.ds(r0, RT)],
                out.at[pl.ds(r0, RT), pl.ds(chunk * width, width)],
            )

    return prop


_prop2 = _make_prop(2, L, jnp.float32)
_prop4b = _make_prop(4, 2 * L, jnp.bfloat16)


@functools.partial(
    pl.kernel,
    out_type=jax.ShapeDtypeStruct((NSC, N, L), jnp.float32),
    mesh=_sc_mesh(),
    scratch_types=[
        pltpu.VMEM_SHARED((N, L), jnp.float32),
        pltpu.VMEM((GEH, 128), jnp.int32),
        pltpu.VMEM((128, L), jnp.float32),
        pltpu.SemaphoreType.DMA,
    ],
    compiler_params=_SC_PARAMS,
)
def _hist(dst2d, zrow, onesr, out, acc1, dbuf, onesb, semS):
    c = lax.axis_index("c")
    sid = lax.axis_index("s")
    r0 = sid * RT
    pltpu.sync_copy(zrow, acc1.at[pl.ds(r0, RT)])
    pltpu.sync_copy(onesr, onesb)
    plsc.subcore_barrier()

    w = c * NSUB + sid

    @pl.loop(w, EG // GEH, step=NSC * NSUB)
    def _(g):
        @pl.when(g != w)
        def _():
            for j in range(GEH):
                pltpu.make_async_copy(onesb, acc1.at[dbuf.at[j]], semS).wait()

        pltpu.sync_copy(dst2d.at[pl.ds(g * GEH, GEH)], dbuf)
        for j in range(GEH):
            pltpu.async_copy(onesb, acc1.at[dbuf.at[j]], semS, add=True)

    for j in range(GEH):
        pltpu.make_async_copy(onesb, acc1.at[dbuf.at[j]], semS).wait()

    plsc.subcore_barrier()
    pltpu.sync_copy(acc1.at[pl.ds(r0, RT)], out.at[c, pl.ds(r0, RT)])


_BN1 = 2000


def _k1_body(x32_ref, d0_ref, d1_ref, t1_ref, sv_ref):
    deg = d0_ref[0][:, 0:1] + d1_ref[0][:, 0:1] + 1.0
    s = lax.rsqrt(deg)
    sv_ref[...] = s
    t1_ref[...] = x32_ref[...] * s


def _k1(x32, degp):
    return pl.pallas_call(
        _k1_body,
        grid=(N // _BN1,),
        in_specs=[
            pl.BlockSpec((_BN1, D_P), lambda i: (i, 0)),
            pl.BlockSpec((1, _BN1, L), lambda i: (0, i, 0)),
            pl.BlockSpec((1, _BN1, L), lambda i: (1, i, 0)),
        ],
        out_specs=[
            pl.BlockSpec((_BN1, D_P), lambda i: (i, 0)),
            pl.BlockSpec((_BN1, 1), lambda i: (i, 0)),
        ],
        out_shape=[
            jax.ShapeDtypeStruct((N, D_P), jnp.float32),
            jax.ShapeDtypeStruct((N, 1), jnp.float32),
        ],
    )(x32, degp, degp)


_BN2 = 2000


def _k2_body(agg_ref, t1_ref, sv_ref, w_ref, b_ref, t2_ref, t2b_ref):
    s = sv_ref[...]
    p = (agg_ref[...] + t1_ref[...]) * s
    h = jnp.maximum(
        jnp.dot(p, w_ref[...], preferred_element_type=jnp.float32) + b_ref[...],
        0.0,
    )
    t2 = h * s
    t2_ref[...] = t2
    t2b_ref[...] = t2.astype(jnp.bfloat16)


def _k2(agg1, t1, sv, W1p, b1r):
    return pl.pallas_call(
        _k2_body,
        grid=(N // _BN2,),
        in_specs=[
            pl.BlockSpec((_BN2, D_P), lambda i: (i, 0)),
            pl.BlockSpec((_BN2, D_P), lambda i: (i, 0)),
            pl.BlockSpec((_BN2, 1), lambda i: (i, 0)),
            pl.BlockSpec((D_P, D_H), lambda i: (0, 0)),
            pl.BlockSpec((1, D_H), lambda i: (0, 0)),
        ],
        out_specs=[
            pl.BlockSpec((_BN2, D_H), lambda i: (i, 0)),
            pl.BlockSpec((_BN2, D_H), lambda i: (i, 0)),
        ],
        out_shape=[
            jax.ShapeDtypeStruct((N, D_H), jnp.float32),
            jax.ShapeDtypeStruct((N, D_H), jnp.bfloat16),
        ],
    )(agg1, t1, sv, W1p, b1r)


def _k3_body(agg_ref, t2_ref, sv_ref, bat_ref, w_ref, b_ref, out_ref, pacc, cacc):
    i = pl.program_id(0)

    @pl.when(i == 0)
    def _():
        pacc[...] = jnp.zeros_like(pacc)
        cacc[...] = jnp.zeros_like(cacc)

    s = sv_ref[...]
    p = (agg_ref[...].astype(jnp.float32) + t2_ref[...]) * s
    h = jnp.maximum(
        jnp.dot(p, w_ref[...], preferred_element_type=jnp.float32) + b_ref[...],
        0.0,
    )
    gids = lax.broadcasted_iota(jnp.int32, (G, _BN2), 0)
    onehot = jnp.where(gids == bat_ref[0], 1.0, 0.0)
    pacc[...] += jnp.dot(onehot, h, preferred_element_type=jnp.float32)
    cacc[...] += jnp.sum(onehot, axis=1, keepdims=True)

    @pl.when(i == pl.num_programs(0) - 1)
    def _():
        out_ref[...] = pacc[...] / jnp.maximum(cacc[...], 1.0)


def _k3(agg2, t2, sv, batchr, W2, b2r):
    return pl.pallas_call(
        _k3_body,
        grid=(N // _BN2,),
        in_specs=[
            pl.BlockSpec((_BN2, D_H), lambda i: (i, 0)),
            pl.BlockSpec((_BN2, D_H), lambda i: (i, 0)),
            pl.BlockSpec((_BN2, 1), lambda i: (i, 0)),
            pl.BlockSpec((1, 1, _BN2), lambda i: (i, 0, 0)),
            pl.BlockSpec((D_H, D_H), lambda i: (0, 0)),
            pl.BlockSpec((1, D_H), lambda i: (0, 0)),
        ],
        out_specs=pl.BlockSpec((G, D_H), lambda i: (0, 0)),
        out_shape=jax.ShapeDtypeStruct((G, D_H), jnp.float32),
        scratch_shapes=[
            pltpu.VMEM((G, D_H), jnp.float32),
            pltpu.VMEM((G, 1), jnp.float32),
        ],
    )(agg2, t2, sv, batchr, W2, b2r)


def _k4_body(p_ref, wc_ref, bc_ref, out_ref):
    out_ref[...] = (
        jnp.dot(p_ref[...], wc_ref[...], preferred_element_type=jnp.float32)
        + bc_ref[...]
    )


def _k4(pooled, Wc, bcr):
    return pl.pallas_call(
        _k4_body,
        out_shape=jax.ShapeDtypeStruct((G, D_OUT), jnp.float32),
    )(pooled, Wc, bcr)


def kernel(x, edge_index, batch, W1, b1, W2, b2, Wc, bc):
    src2d = edge_index[0].reshape(EG, 128)
    dst2d = edge_index[1].reshape(EG, 128)
    x32 = jnp.pad(x, ((0, 0), (0, D_P - D_IN)))
    W1p = jnp.pad(W1, ((0, D_P - D_IN), (0, 0)))
    b1r = b1.reshape(1, D_H)
    b2r = b2.reshape(1, D_H)
    bcr = bc.reshape(1, D_OUT)
    batchr = batch.reshape(N // _BN2, 1, _BN2)
    zrow = jnp.zeros((RT, L), jnp.float32)
    zrowb = jnp.zeros((RT, 2 * L), jnp.bfloat16)
    onesr = jnp.ones((128, L), jnp.float32)

    degp = _hist(dst2d, zrow, onesr)
    t1, sv = _k1(x32, degp)
    agg1 = _prop2(t1.reshape(N * 2, L), src2d, dst2d, zrow)
    t2, t2b = _k2(agg1, t1, sv, W1p, b1r)
    agg2 = _prop4b(t2b.reshape(N * 4, 2 * L), src2d, dst2d, zrowb)
    pooled = _k3(agg2, t2, sv, batchr, W2, b2r)
    return _k4(pooled, Wc, bcr)
